# Initial kernel scaffold; baseline (speedup 1.0000x reference)
import jax
import jax.numpy as jnp
from jax.experimental import pallas as pl


def kernel(x, edge_index, W, b):
    raise NotImplementedError("write your pallas kernel here")


# speedup vs baseline: 22.1358x; 22.1358x over previous
import functools

import jax
import jax.numpy as jnp
from jax import lax
from jax.experimental import pallas as pl
from jax.experimental.pallas import tpu as pltpu
from jax.experimental.pallas import tpu_sc as plsc

N = 10000
E = 320000
D = 128

NC, NS = 2, 16
NW = NC * NS
CHUNK = 128
NCHUNK = E // CHUNK
ITERS = (NCHUNK + NW - 1) // NW
NPAD = 10240
DEG_PER_TILE = NPAD // NS
ROWS_PER_TILE = NPAD // NS

_mesh = plsc.VectorSubcoreMesh(core_axis_name="c", subcore_axis_name="s")


@functools.partial(
    pl.kernel,
    out_type=jax.ShapeDtypeStruct((NC * NPAD,), jnp.float32),
    mesh=_mesh,
    scratch_types=[
        pltpu.VMEM((CHUNK,), jnp.int32),
        pltpu.VMEM((CHUNK,), jnp.float32),
        pltpu.VMEM((DEG_PER_TILE,), jnp.float32),
        pltpu.VMEM_SHARED((NPAD,), jnp.float32),
        pltpu.SemaphoreType.DMA,
    ],
)
def _deg_kernel(dst_hbm, out_hbm, idx_v, ones_v, buf_v, acc_sh, sem):
    del sem
    cid = lax.axis_index("c")
    sid = lax.axis_index("s")
    wid = sid * NC + cid

    for j in range(CHUNK // 16):
        ones_v[pl.ds(j * 16, 16)] = jnp.ones((16,), jnp.float32)

    def _zero(i, carry):
        buf_v[pl.ds(i * 16, 16)] = jnp.zeros((16,), jnp.float32)
        return carry

    lax.fori_loop(0, DEG_PER_TILE // 16, _zero, 0)
    pltpu.sync_copy(buf_v, acc_sh.at[pl.ds(sid * DEG_PER_TILE, DEG_PER_TILE)])
    plsc.subcore_barrier()

    def _body(k, carry):
        c = wid + NW * k

        @pl.when(c < NCHUNK)
        def _():
            pltpu.sync_copy(dst_hbm.at[pl.ds(c * CHUNK, CHUNK)], idx_v)
            pltpu.sync_copy(ones_v, acc_sh.at[idx_v], add=True)

        return carry

    lax.fori_loop(0, ITERS, _body, 0)
    plsc.subcore_barrier()

    pltpu.sync_copy(acc_sh.at[pl.ds(sid * DEG_PER_TILE, DEG_PER_TILE)], buf_v)
    pltpu.sync_copy(
        buf_v, out_hbm.at[pl.ds(cid * NPAD + sid * DEG_PER_TILE, DEG_PER_TILE)]
    )


@functools.partial(
    pl.kernel,
    out_type=jax.ShapeDtypeStruct((NC * NPAD, D), jnp.float32),
    mesh=_mesh,
    scratch_types=[
        pltpu.VMEM((CHUNK,), jnp.int32),
        pltpu.VMEM((CHUNK,), jnp.int32),
        pltpu.VMEM((CHUNK, D), jnp.float32),
        pltpu.VMEM((CHUNK, D), jnp.float32),
        pltpu.VMEM_SHARED((NPAD, D), jnp.float32),
        pltpu.SemaphoreType.DMA,
    ],
)
def _scatter_kernel(src_hbm, dst_hbm, hp_hbm, out_hbm, si_v, di_v, rows_v,
                    buf_v, acc_sh, sem):
    cid = lax.axis_index("c")
    sid = lax.axis_index("s")
    wid = sid * NC + cid

    def _zero(i, carry):
        for j in range(D // 16):
            buf_v[i, pl.ds(j * 16, 16)] = jnp.zeros((16,), jnp.float32)
        return carry

    lax.fori_loop(0, CHUNK, _zero, 0)

    r0 = sid * ROWS_PER_TILE
    for t in range(ROWS_PER_TILE // CHUNK):
        pltpu.sync_copy(buf_v, acc_sh.at[pl.ds(r0 + t * CHUNK, CHUNK)])
    plsc.subcore_barrier()

    def _body(k, carry):
        c = wid + NW * k

        @pl.when(c < NCHUNK)
        def _():
            pltpu.sync_copy(src_hbm.at[pl.ds(c * CHUNK, CHUNK)], si_v)
            pltpu.sync_copy(dst_hbm.at[pl.ds(c * CHUNK, CHUNK)], di_v)
            pltpu.async_copy(hp_hbm.at[si_v], rows_v, sem).wait()
            pltpu.sync_copy(rows_v, acc_sh.at[di_v], add=True)

        return carry

    lax.fori_loop(0, ITERS, _body, 0)
    plsc.subcore_barrier()

    for t in range(ROWS_PER_TILE // CHUNK):
        pltpu.sync_copy(acc_sh.at[pl.ds(r0 + t * CHUNK, CHUNK)], buf_v)
        pltpu.sync_copy(buf_v,
                        out_hbm.at[pl.ds(cid * NPAD + r0 + t * CHUNK, CHUNK)])


MB = 1000


def _mm_body(x_ref, w_ref, deg_ref, hp_ref, dis_ref):
    dis = lax.rsqrt(deg_ref[...])
    h = jnp.dot(x_ref[...], w_ref[...], preferred_element_type=jnp.float32)
    hp_ref[...] = h * dis
    dis_ref[...] = dis


_mm_call = pl.pallas_call(
    _mm_body,
    grid=(N // MB,),
    in_specs=[
        pl.BlockSpec((MB, D), lambda i: (i, 0)),
        pl.BlockSpec((D, D), lambda i: (0, 0)),
        pl.BlockSpec((MB, 1), lambda i: (i, 0)),
    ],
    out_specs=[
        pl.BlockSpec((MB, D), lambda i: (i, 0)),
        pl.BlockSpec((MB, 1), lambda i: (i, 0)),
    ],
    out_shape=[
        jax.ShapeDtypeStruct((N, D), jnp.float32),
        jax.ShapeDtypeStruct((N, 1), jnp.float32),
    ],
)


def _final_body(acc_ref, hp_ref, dis_ref, b_ref, o_ref):
    s = acc_ref[0] + acc_ref[1] + hp_ref[...]
    o_ref[...] = jnp.maximum(s * dis_ref[...] + b_ref[...], 0.0)


_final_call = pl.pallas_call(
    _final_body,
    grid=(N // MB,),
    in_specs=[
        pl.BlockSpec((NC, MB, D), lambda i: (0, i, 0)),
        pl.BlockSpec((MB, D), lambda i: (i, 0)),
        pl.BlockSpec((MB, 1), lambda i: (i, 0)),
        pl.BlockSpec((1, D), lambda i: (0, 0)),
    ],
    out_specs=pl.BlockSpec((MB, D), lambda i: (i, 0)),
    out_shape=jax.ShapeDtypeStruct((N, D), jnp.float32),
)


@jax.jit
def kernel(x, edge_index, W, b):
    src = edge_index[0].astype(jnp.int32)
    dst = edge_index[1].astype(jnp.int32)

    degp = _deg_kernel(dst)
    deg = (1.0 + degp[:N] + degp[NPAD:NPAD + N]).reshape(N, 1)

    hp, dis = _mm_call(x, W, deg)

    acc = _scatter_kernel(src, dst, hp).reshape(NC, NPAD, D)

    return _final_call(acc, hp, dis, b.reshape(1, D))
